# baseline (speedup 1.0000x reference)
import functools
import math

import jax
import jax.numpy as jnp
from jax import lax
from jax.experimental import pallas as pl
from jax.experimental.pallas import tpu as pltpu
from jax.experimental.pallas import tpu_sc as plsc

_B, _N, _C = 4, 4096, 1024
_K = math.floor(0.75 * _N)

_BI = 512
_NB = _N // _BI

_NS = 16
_TPW = _N // _NS
_RPW = _K // _NS
_CH = 48
_NCH = _RPW // _CH
_NBUF = 2


def _rank_body(s_ref, out_ref):
    s_row = s_ref[0, 0:1, :]
    s_col = s_row.reshape(_N, 1)
    jlt = (lax.broadcasted_iota(jnp.int32, (_BI, _BI), 0)
           < lax.broadcasted_iota(jnp.int32, (_BI, _BI), 1))
    for ic in range(_NB):
        si = s_row[0:1, ic * _BI:(ic + 1) * _BI]
        acc = jnp.zeros((1, _BI), jnp.int32)
        for jc in range(_NB):
            sj = s_col[jc * _BI:(jc + 1) * _BI, 0:1]
            if jc < ic:
                cmp = sj >= si
            elif jc > ic:
                cmp = sj > si
            else:
                cmp = (sj > si) | ((sj == si) & jlt)
            acc = acc + jnp.sum(cmp.astype(jnp.int32), axis=0, keepdims=True)
        out_ref[0, 0:1, ic * _BI:(ic + 1) * _BI] = acc


_rank_call = pl.pallas_call(
    _rank_body,
    grid=(2,),
    in_specs=[pl.BlockSpec((1, 1, _N), lambda b: (b, 0, 0))],
    out_specs=pl.BlockSpec((1, 1, _N), lambda b: (b, 0, 0)),
    out_shape=jax.ShapeDtypeStruct((2, 1, _N), jnp.int32),
)


def _sc_prune_body(gbase, ranks_hbm, x_hbm, m_hbm, xout_hbm, mout_hbm,
                   didx1_v, didx_v, vals_v, perm_sh, ridx_v, mout_v, *bufsems):
    bufs = bufsems[:_NBUF]
    gsems = bufsems[_NBUF:2 * _NBUF]
    ssems = bufsems[2 * _NBUF:3 * _NBUF]
    sem_m = bufsems[3 * _NBUF]
    c = lax.axis_index("c")
    s = lax.axis_index("s")

    with jax.named_scope("partA"):
        tok0 = c * _N + s * _TPW
        pltpu.sync_copy(ranks_hbm.at[pl.ds(tok0, _TPW)], didx1_v)
        for row in range(_TPW // 128):
            for cc in range(128 // 16):
                didx_v.at[row][pl.ds(cc * 16, 16)] = (
                    didx1_v[pl.ds(row * 128 + cc * 16, 16)])
                vals_v.at[row][pl.ds(cc * 16, 16)] = (
                    gbase + tok0 + row * 128 + cc * 16 + lax.iota(jnp.int32, 16))
        for row in range(_TPW // 128):
            pltpu.sync_copy(vals_v.at[row], perm_sh.at[didx_v.at[row]])

    with jax.named_scope("barrier"):
        plsc.subcore_barrier()

    with jax.named_scope("permload"):
        p0 = s * _RPW
        pltpu.sync_copy(perm_sh.at[pl.ds(p0, _RPW)], ridx_v)

    out0 = c * _K + s * _RPW

    hm = [pltpu.async_copy(m_hbm.at[ridx_v.at[pl.ds(j * 96, 96)]],
                           mout_v.at[pl.ds(j * 96, 96)], sem_m)
          for j in range(_RPW // 96)]

    lag = _NBUF - 1
    hg = [None] * _NCH
    hs = [None] * _NCH

    def _store(c2):
        s2 = c2 % _NBUF
        hg[c2].wait()
        hs[c2] = pltpu.async_copy(
            bufs[s2], xout_hbm.at[pl.ds(out0 + c2 * _CH, _CH)], ssems[s2])

    with jax.named_scope("xloop"):
        for ch in range(_NCH):
            slot = ch % _NBUF
            if ch >= _NBUF:
                hs[ch - _NBUF].wait()
            hg[ch] = pltpu.async_copy(
                x_hbm.at[ridx_v.at[pl.ds(ch * _CH, _CH)]], bufs[slot], gsems[slot])
            if ch >= lag:
                _store(ch - lag)
        for c2 in range(_NCH - lag, _NCH):
            _store(c2)
        for c2 in range(_NCH - _NBUF, _NCH):
            hs[c2].wait()

    with jax.named_scope("mask"):
        for h in hm:
            h.wait()
        pltpu.sync_copy(mout_v, mout_hbm.at[pl.ds(out0, _RPW)])


@functools.cache
def _build_sc_prune(pair):
    return pl.kernel(
        functools.partial(_sc_prune_body, pair * 2 * _N),
        mesh=plsc.VectorSubcoreMesh(core_axis_name="c", subcore_axis_name="s"),
        out_type=(
            jax.ShapeDtypeStruct((2 * _K, _C), jnp.float32),
            jax.ShapeDtypeStruct((2 * _K,), jnp.float32),
        ),
        scratch_types=[
            pltpu.VMEM((_TPW,), jnp.int32),
            pltpu.VMEM((_TPW // 128, 128), jnp.int32),
            pltpu.VMEM((_TPW // 128, 128), jnp.int32),
            pltpu.VMEM_SHARED((_N,), jnp.int32),
            pltpu.VMEM((_RPW,), jnp.int32),
            pltpu.VMEM((_RPW,), jnp.float32),
        ] + [pltpu.VMEM((_CH, _C), jnp.float32) for _ in range(_NBUF)]
          + [pltpu.SemaphoreType.DMA for _ in range(2 * _NBUF + 1)],
    )


def kernel(x, m, scores):
    x_flat = x.reshape(_B * _N, _C)
    m_flat = m.reshape(_B * _N)
    r01 = _rank_call(scores[:2].reshape(2, 1, _N)).reshape(2 * _N)
    r23 = _rank_call(scores[2:].reshape(2, 1, _N)).reshape(2 * _N)
    xo01, mo01 = _build_sc_prune(0)(r01, x_flat, m_flat)
    xo23, mo23 = _build_sc_prune(1)(r23, x_flat, m_flat)
    x_out = jnp.concatenate(
        [xo01.reshape(2, _K, _C), xo23.reshape(2, _K, _C)], axis=0)
    m_out = jnp.concatenate([mo01, mo23]).reshape(_B, 1, 1, _K)
    return x_out, m_out


# speedup vs baseline: 1.4176x; 1.4176x over previous
import functools
import math

import jax
import jax.numpy as jnp
from jax import lax
from jax.experimental import pallas as pl
from jax.experimental.pallas import tpu as pltpu
from jax.experimental.pallas import tpu_sc as plsc

_B, _N, _C = 4, 4096, 1024
_K = math.floor(0.75 * _N)

_BI = 512
_NB = _N // _BI

_NS = 16
_TPW = _N // _NS
_RPW = _K // _NS
_CH = 48
_NCH = _RPW // _CH
_NBUF = 2


def _rank_body(s_ref, out_ref):
    s_row = s_ref[0, 0:1, :]
    s_col = s_row.reshape(_N, 1)
    jlt = (lax.broadcasted_iota(jnp.int32, (_BI, _BI), 0)
           < lax.broadcasted_iota(jnp.int32, (_BI, _BI), 1))
    for ic in range(_NB):
        si = s_row[0:1, ic * _BI:(ic + 1) * _BI]
        acc = jnp.zeros((1, _BI), jnp.int32)
        for jc in range(_NB):
            sj = s_col[jc * _BI:(jc + 1) * _BI, 0:1]
            if jc < ic:
                cmp = sj >= si
            elif jc > ic:
                cmp = sj > si
            else:
                cmp = (sj > si) | ((sj == si) & jlt)
            acc = acc + jnp.sum(cmp.astype(jnp.int32), axis=0, keepdims=True)
        out_ref[0, 0:1, ic * _BI:(ic + 1) * _BI] = acc


_rank_call = pl.pallas_call(
    _rank_body,
    grid=(2,),
    in_specs=[pl.BlockSpec((1, 1, _N), lambda b: (b, 0, 0))],
    out_specs=pl.BlockSpec((1, 1, _N), lambda b: (b, 0, 0)),
    out_shape=jax.ShapeDtypeStruct((2, 1, _N), jnp.int32),
)


def _sc_prune_body(gbase, ranks_hbm, x_hbm, m_hbm, xout_hbm, mout_hbm,
                   didx1_v, didx_v, vals_v, perm_sh, ridx_v, mout_v, *bufsems):
    bufs = bufsems[:_NBUF]
    gsems = bufsems[_NBUF:2 * _NBUF]
    ssems = bufsems[2 * _NBUF:3 * _NBUF]
    sem_m = bufsems[3 * _NBUF]
    c = lax.axis_index("c")
    s = lax.axis_index("s")

    with jax.named_scope("partA"):
        tok0 = c * _N + s * _TPW
        pltpu.sync_copy(ranks_hbm.at[pl.ds(tok0, _TPW)], didx1_v)
        for row in range(_TPW // 128):
            for cc in range(128 // 16):
                didx_v.at[row][pl.ds(cc * 16, 16)] = (
                    didx1_v[pl.ds(row * 128 + cc * 16, 16)])
                vals_v.at[row][pl.ds(cc * 16, 16)] = (
                    gbase + tok0 + row * 128 + cc * 16 + lax.iota(jnp.int32, 16))
        for row in range(_TPW // 128):
            pltpu.sync_copy(vals_v.at[row], perm_sh.at[didx_v.at[row]])

    with jax.named_scope("barrier"):
        plsc.subcore_barrier()

    with jax.named_scope("permload"):
        p0 = s * _RPW
        pltpu.sync_copy(perm_sh.at[pl.ds(p0, _RPW)], ridx_v)

    out0 = gbase // _N * _K + c * _K + s * _RPW

    hm = [pltpu.async_copy(m_hbm.at[ridx_v.at[pl.ds(j * 96, 96)]],
                           mout_v.at[pl.ds(j * 96, 96)], sem_m)
          for j in range(_RPW // 96)]

    lag = _NBUF - 1
    hg = [None] * _NCH
    hs = [None] * _NCH

    def _store(c2):
        s2 = c2 % _NBUF
        hg[c2].wait()
        hs[c2] = pltpu.async_copy(
            bufs[s2], xout_hbm.at[pl.ds(out0 + c2 * _CH, _CH)], ssems[s2])

    with jax.named_scope("xloop"):
        for ch in range(_NCH):
            slot = ch % _NBUF
            if ch >= _NBUF:
                hs[ch - _NBUF].wait()
            hg[ch] = pltpu.async_copy(
                x_hbm.at[ridx_v.at[pl.ds(ch * _CH, _CH)]], bufs[slot], gsems[slot])
            if ch >= lag:
                _store(ch - lag)
        for c2 in range(_NCH - lag, _NCH):
            _store(c2)
        for c2 in range(_NCH - _NBUF, _NCH):
            hs[c2].wait()

    with jax.named_scope("mask"):
        for h in hm:
            h.wait()
        pltpu.sync_copy(mout_v, mout_hbm.at[pl.ds(out0, _RPW)])


@functools.cache
def _build_sc_prune(pair):
    return pl.kernel(
        functools.partial(_sc_prune_body, pair * 2 * _N),
        mesh=plsc.VectorSubcoreMesh(core_axis_name="c", subcore_axis_name="s"),
        out_type=(),
        scratch_types=[
            pltpu.VMEM((_TPW,), jnp.int32),
            pltpu.VMEM((_TPW // 128, 128), jnp.int32),
            pltpu.VMEM((_TPW // 128, 128), jnp.int32),
            pltpu.VMEM_SHARED((_N,), jnp.int32),
            pltpu.VMEM((_RPW,), jnp.int32),
            pltpu.VMEM((_RPW,), jnp.float32),
        ] + [pltpu.VMEM((_CH, _C), jnp.float32) for _ in range(_NBUF)]
          + [pltpu.SemaphoreType.DMA for _ in range(2 * _NBUF + 1)],
    )


def kernel(x, m, scores):
    x_flat = x.reshape(_B * _N, _C)
    m_flat = m.reshape(_B * _N)
    r01 = _rank_call(scores[:2].reshape(2, 1, _N)).reshape(2 * _N)
    r23 = _rank_call(scores[2:].reshape(2, 1, _N)).reshape(2 * _N)
    xout_ref = pl.empty_ref_like(pltpu.HBM((_B * _K, _C), jnp.float32))
    mout_ref = pl.empty_ref_like(pltpu.HBM((_B * _K,), jnp.float32))
    _build_sc_prune(0)(r01, x_flat, m_flat, xout_ref, mout_ref)
    _build_sc_prune(1)(r23, x_flat, m_flat, xout_ref, mout_ref)
    return (xout_ref[...].reshape(_B, _K, _C),
            mout_ref[...].reshape(_B, 1, 1, _K))


# speedup vs baseline: 1.4339x; 1.0115x over previous
import functools
import math

import jax
import jax.numpy as jnp
from jax import lax
from jax.experimental import pallas as pl
from jax.experimental.pallas import tpu as pltpu
from jax.experimental.pallas import tpu_sc as plsc

_B, _N, _C = 4, 4096, 1024
_K = math.floor(0.75 * _N)

_BI = 512
_NB = _N // _BI

_NS = 16
_TPW = _N // _NS
_RPW = _K // _NS
_CH = 24
_NCH = _RPW // _CH
_NBUF = 4


def _rank_body(s_ref, out_ref):
    s_row = s_ref[0, 0:1, :]
    s_col = s_row.reshape(_N, 1)
    jlt = (lax.broadcasted_iota(jnp.int32, (_BI, _BI), 0)
           < lax.broadcasted_iota(jnp.int32, (_BI, _BI), 1))
    for ic in range(_NB):
        si = s_row[0:1, ic * _BI:(ic + 1) * _BI]
        acc = jnp.zeros((1, _BI), jnp.int32)
        for jc in range(_NB):
            sj = s_col[jc * _BI:(jc + 1) * _BI, 0:1]
            if jc < ic:
                cmp = sj >= si
            elif jc > ic:
                cmp = sj > si
            else:
                cmp = (sj > si) | ((sj == si) & jlt)
            acc = acc + jnp.sum(cmp.astype(jnp.int32), axis=0, keepdims=True)
        out_ref[0, 0:1, ic * _BI:(ic + 1) * _BI] = acc


_rank_call = pl.pallas_call(
    _rank_body,
    grid=(2,),
    in_specs=[pl.BlockSpec((1, 1, _N), lambda b: (b, 0, 0))],
    out_specs=pl.BlockSpec((1, 1, _N), lambda b: (b, 0, 0)),
    out_shape=jax.ShapeDtypeStruct((2, 1, _N), jnp.int32),
)


def _sc_prune_body(gbase, ranks_hbm, x_hbm, m_hbm, xout_hbm, mout_hbm,
                   didx1_v, didx_v, vals_v, perm_sh, ridx_v, mout_v, *bufsems):
    bufs = bufsems[:_NBUF]
    gsems = bufsems[_NBUF:2 * _NBUF]
    ssems = bufsems[2 * _NBUF:3 * _NBUF]
    sem_m = bufsems[3 * _NBUF]
    c = lax.axis_index("c")
    s = lax.axis_index("s")

    with jax.named_scope("partA"):
        tok0 = c * _N + s * _TPW
        pltpu.sync_copy(ranks_hbm.at[pl.ds(tok0, _TPW)], didx1_v)
        for row in range(_TPW // 128):
            for cc in range(128 // 16):
                didx_v.at[row][pl.ds(cc * 16, 16)] = (
                    didx1_v[pl.ds(row * 128 + cc * 16, 16)])
                vals_v.at[row][pl.ds(cc * 16, 16)] = (
                    gbase + tok0 + row * 128 + cc * 16 + lax.iota(jnp.int32, 16))
        for row in range(_TPW // 128):
            pltpu.sync_copy(vals_v.at[row], perm_sh.at[didx_v.at[row]])

    with jax.named_scope("barrier"):
        plsc.subcore_barrier()

    with jax.named_scope("permload"):
        p0 = s * _RPW
        pltpu.sync_copy(perm_sh.at[pl.ds(p0, _RPW)], ridx_v)

    out0 = gbase // _N * _K + c * _K + s * _RPW

    hm = [pltpu.async_copy(m_hbm.at[ridx_v.at[pl.ds(j * 96, 96)]],
                           mout_v.at[pl.ds(j * 96, 96)], sem_m)
          for j in range(_RPW // 96)]

    lag = _NBUF - 1
    hg = [None] * _NCH
    hs = [None] * _NCH

    def _store(c2):
        s2 = c2 % _NBUF
        hg[c2].wait()
        hs[c2] = pltpu.async_copy(
            bufs[s2], xout_hbm.at[pl.ds(out0 + c2 * _CH, _CH)], ssems[s2])

    with jax.named_scope("xloop"):
        for ch in range(_NCH):
            slot = ch % _NBUF
            if ch >= _NBUF:
                hs[ch - _NBUF].wait()
            hg[ch] = pltpu.async_copy(
                x_hbm.at[ridx_v.at[pl.ds(ch * _CH, _CH)]], bufs[slot], gsems[slot])
            if ch >= lag:
                _store(ch - lag)
        for c2 in range(_NCH - lag, _NCH):
            _store(c2)
        for c2 in range(_NCH - _NBUF, _NCH):
            hs[c2].wait()

    with jax.named_scope("mask"):
        for h in hm:
            h.wait()
        pltpu.sync_copy(mout_v, mout_hbm.at[pl.ds(out0, _RPW)])


@functools.cache
def _build_sc_prune(pair):
    return pl.kernel(
        functools.partial(_sc_prune_body, pair * 2 * _N),
        mesh=plsc.VectorSubcoreMesh(core_axis_name="c", subcore_axis_name="s"),
        out_type=(),
        scratch_types=[
            pltpu.VMEM((_TPW,), jnp.int32),
            pltpu.VMEM((_TPW // 128, 128), jnp.int32),
            pltpu.VMEM((_TPW // 128, 128), jnp.int32),
            pltpu.VMEM_SHARED((_N,), jnp.int32),
            pltpu.VMEM((_RPW,), jnp.int32),
            pltpu.VMEM((_RPW,), jnp.float32),
        ] + [pltpu.VMEM((_CH, _C), jnp.float32) for _ in range(_NBUF)]
          + [pltpu.SemaphoreType.DMA for _ in range(2 * _NBUF + 1)],
    )


def kernel(x, m, scores):
    x_flat = x.reshape(_B * _N, _C)
    m_flat = m.reshape(_B * _N)
    r01 = _rank_call(scores[:2].reshape(2, 1, _N)).reshape(2 * _N)
    r23 = _rank_call(scores[2:].reshape(2, 1, _N)).reshape(2 * _N)
    xout_ref = pl.empty_ref_like(pltpu.HBM((_B * _K, _C), jnp.float32))
    mout_ref = pl.empty_ref_like(pltpu.HBM((_B * _K,), jnp.float32))
    _build_sc_prune(0)(r01, x_flat, m_flat, xout_ref, mout_ref)
    _build_sc_prune(1)(r23, x_flat, m_flat, xout_ref, mout_ref)
    return (xout_ref[...].reshape(_B, _K, _C),
            mout_ref[...].reshape(_B, 1, 1, _K))


# speedup vs baseline: 1.4495x; 1.0109x over previous
import functools
import math

import jax
import jax.numpy as jnp
from jax import lax
from jax.experimental import pallas as pl
from jax.experimental.pallas import tpu as pltpu
from jax.experimental.pallas import tpu_sc as plsc

_B, _N, _C = 4, 4096, 1024
_K = math.floor(0.75 * _N)

_BI = 512
_NB = _N // _BI

_NS = 16
_TPW = _N // _NS
_RPW = _K // _NS
_CH = 16
_NCH = _RPW // _CH
_NBUF = 6


def _rank_body(s_ref, out_ref):
    s_row = s_ref[0, 0:1, :]
    s_col = s_row.reshape(_N, 1)
    jlt = (lax.broadcasted_iota(jnp.int32, (_BI, _BI), 0)
           < lax.broadcasted_iota(jnp.int32, (_BI, _BI), 1))
    for ic in range(_NB):
        si = s_row[0:1, ic * _BI:(ic + 1) * _BI]
        acc = jnp.zeros((1, _BI), jnp.int32)
        for jc in range(_NB):
            sj = s_col[jc * _BI:(jc + 1) * _BI, 0:1]
            if jc < ic:
                cmp = sj >= si
            elif jc > ic:
                cmp = sj > si
            else:
                cmp = (sj > si) | ((sj == si) & jlt)
            acc = acc + jnp.sum(cmp.astype(jnp.int32), axis=0, keepdims=True)
        out_ref[0, 0:1, ic * _BI:(ic + 1) * _BI] = acc


_rank_call = pl.pallas_call(
    _rank_body,
    grid=(2,),
    in_specs=[pl.BlockSpec((1, 1, _N), lambda b: (b, 0, 0))],
    out_specs=pl.BlockSpec((1, 1, _N), lambda b: (b, 0, 0)),
    out_shape=jax.ShapeDtypeStruct((2, 1, _N), jnp.int32),
)


def _sc_prune_body(gbase, ranks_hbm, x_hbm, m_hbm, xout_hbm, mout_hbm,
                   didx1_v, didx_v, vals_v, perm_sh, ridx_v, mout_v, *bufsems):
    bufs = bufsems[:_NBUF]
    gsems = bufsems[_NBUF:2 * _NBUF]
    ssems = bufsems[2 * _NBUF:3 * _NBUF]
    sem_m = bufsems[3 * _NBUF]
    c = lax.axis_index("c")
    s = lax.axis_index("s")

    with jax.named_scope("partA"):
        tok0 = c * _N + s * _TPW
        pltpu.sync_copy(ranks_hbm.at[pl.ds(tok0, _TPW)], didx1_v)
        for row in range(_TPW // 128):
            for cc in range(128 // 16):
                didx_v.at[row][pl.ds(cc * 16, 16)] = (
                    didx1_v[pl.ds(row * 128 + cc * 16, 16)])
                vals_v.at[row][pl.ds(cc * 16, 16)] = (
                    gbase + tok0 + row * 128 + cc * 16 + lax.iota(jnp.int32, 16))
        for row in range(_TPW // 128):
            pltpu.sync_copy(vals_v.at[row], perm_sh.at[didx_v.at[row]])

    with jax.named_scope("barrier"):
        plsc.subcore_barrier()

    with jax.named_scope("permload"):
        p0 = s * _RPW
        pltpu.sync_copy(perm_sh.at[pl.ds(p0, _RPW)], ridx_v)

    out0 = gbase // _N * _K + c * _K + s * _RPW

    hm = [pltpu.async_copy(m_hbm.at[ridx_v.at[pl.ds(j * 96, 96)]],
                           mout_v.at[pl.ds(j * 96, 96)], sem_m)
          for j in range(_RPW // 96)]

    lag = _NBUF - 1
    hg = [None] * _NCH
    hs = [None] * _NCH

    def _store(c2):
        s2 = c2 % _NBUF
        hg[c2].wait()
        hs[c2] = pltpu.async_copy(
            bufs[s2], xout_hbm.at[pl.ds(out0 + c2 * _CH, _CH)], ssems[s2])

    with jax.named_scope("xloop"):
        for ch in range(_NCH):
            slot = ch % _NBUF
            if ch >= _NBUF:
                hs[ch - _NBUF].wait()
            hg[ch] = pltpu.async_copy(
                x_hbm.at[ridx_v.at[pl.ds(ch * _CH, _CH)]], bufs[slot], gsems[slot])
            if ch >= lag:
                _store(ch - lag)
        for c2 in range(_NCH - lag, _NCH):
            _store(c2)
        for c2 in range(_NCH - _NBUF, _NCH):
            hs[c2].wait()

    with jax.named_scope("mask"):
        for h in hm:
            h.wait()
        pltpu.sync_copy(mout_v, mout_hbm.at[pl.ds(out0, _RPW)])


@functools.cache
def _build_sc_prune(pair):
    return pl.kernel(
        functools.partial(_sc_prune_body, pair * 2 * _N),
        mesh=plsc.VectorSubcoreMesh(core_axis_name="c", subcore_axis_name="s"),
        out_type=(),
        scratch_types=[
            pltpu.VMEM((_TPW,), jnp.int32),
            pltpu.VMEM((_TPW // 128, 128), jnp.int32),
            pltpu.VMEM((_TPW // 128, 128), jnp.int32),
            pltpu.VMEM_SHARED((_N,), jnp.int32),
            pltpu.VMEM((_RPW,), jnp.int32),
            pltpu.VMEM((_RPW,), jnp.float32),
        ] + [pltpu.VMEM((_CH, _C), jnp.float32) for _ in range(_NBUF)]
          + [pltpu.SemaphoreType.DMA for _ in range(2 * _NBUF + 1)],
    )


def kernel(x, m, scores):
    x_flat = x.reshape(_B * _N, _C)
    m_flat = m.reshape(_B * _N)
    r01 = _rank_call(scores[:2].reshape(2, 1, _N)).reshape(2 * _N)
    r23 = _rank_call(scores[2:].reshape(2, 1, _N)).reshape(2 * _N)
    xout_ref = pl.empty_ref_like(pltpu.HBM((_B * _K, _C), jnp.float32))
    mout_ref = pl.empty_ref_like(pltpu.HBM((_B * _K,), jnp.float32))
    _build_sc_prune(0)(r01, x_flat, m_flat, xout_ref, mout_ref)
    _build_sc_prune(1)(r23, x_flat, m_flat, xout_ref, mout_ref)
    return (xout_ref[...].reshape(_B, _K, _C),
            mout_ref[...].reshape(_B, 1, 1, _K))


# speedup vs baseline: 1.4507x; 1.0008x over previous
import functools
import math

import jax
import jax.numpy as jnp
from jax import lax
from jax.experimental import pallas as pl
from jax.experimental.pallas import tpu as pltpu
from jax.experimental.pallas import tpu_sc as plsc

_B, _N, _C = 4, 4096, 1024
_K = math.floor(0.75 * _N)

_BI = 512
_NB = _N // _BI

_NS = 16
_TPW = _N // _NS
_RPW = _K // _NS
_CH = 16
_NCH = _RPW // _CH
_NBUF = 6


def _rank_body(s_ref, out_ref):
    jlt = (lax.broadcasted_iota(jnp.int32, (_BI, _BI), 0)
           < lax.broadcasted_iota(jnp.int32, (_BI, _BI), 1))
    for b in range(2):
        s_row = s_ref[b, 0:1, :]
        s_col = s_row.reshape(_N, 1)
        for ic in range(_NB):
            si = s_row[0:1, ic * _BI:(ic + 1) * _BI]
            acc = jnp.zeros((1, _BI), jnp.int32)
            for jc in range(_NB):
                sj = s_col[jc * _BI:(jc + 1) * _BI, 0:1]
                if jc < ic:
                    cmp = sj >= si
                elif jc > ic:
                    cmp = sj > si
                else:
                    cmp = (sj > si) | ((sj == si) & jlt)
                acc = acc + jnp.sum(cmp.astype(jnp.int32), axis=0, keepdims=True)
            out_ref[b, 0:1, ic * _BI:(ic + 1) * _BI] = acc


_rank_call = pl.pallas_call(
    _rank_body,
    out_shape=jax.ShapeDtypeStruct((2, 1, _N), jnp.int32),
)


def _sc_prune_body(gbase, ranks_hbm, x_hbm, m_hbm, xout_hbm, mout_hbm,
                   didx1_v, didx_v, vals_v, perm_sh, ridx_v, mout_v, *bufsems):
    bufs = bufsems[:_NBUF]
    gsems = bufsems[_NBUF:2 * _NBUF]
    ssems = bufsems[2 * _NBUF:3 * _NBUF]
    sem_m = bufsems[3 * _NBUF]
    c = lax.axis_index("c")
    s = lax.axis_index("s")

    with jax.named_scope("partA"):
        tok0 = c * _N + s * _TPW
        pltpu.sync_copy(ranks_hbm.at[pl.ds(tok0, _TPW)], didx1_v)
        for row in range(_TPW // 128):
            for cc in range(128 // 16):
                didx_v.at[row][pl.ds(cc * 16, 16)] = (
                    didx1_v[pl.ds(row * 128 + cc * 16, 16)])
                vals_v.at[row][pl.ds(cc * 16, 16)] = (
                    gbase + tok0 + row * 128 + cc * 16 + lax.iota(jnp.int32, 16))
        for row in range(_TPW // 128):
            pltpu.sync_copy(vals_v.at[row], perm_sh.at[didx_v.at[row]])

    with jax.named_scope("barrier"):
        plsc.subcore_barrier()

    with jax.named_scope("permload"):
        p0 = s * _RPW
        pltpu.sync_copy(perm_sh.at[pl.ds(p0, _RPW)], ridx_v)

    out0 = gbase // _N * _K + c * _K + s * _RPW

    hm = [pltpu.async_copy(m_hbm.at[ridx_v.at[pl.ds(j * 96, 96)]],
                           mout_v.at[pl.ds(j * 96, 96)], sem_m)
          for j in range(_RPW // 96)]

    lag = _NBUF - 1
    hg = [None] * _NCH
    hs = [None] * _NCH

    def _store(c2):
        s2 = c2 % _NBUF
        hg[c2].wait()
        hs[c2] = pltpu.async_copy(
            bufs[s2], xout_hbm.at[pl.ds(out0 + c2 * _CH, _CH)], ssems[s2])

    with jax.named_scope("xloop"):
        for ch in range(_NCH):
            slot = ch % _NBUF
            if ch >= _NBUF:
                hs[ch - _NBUF].wait()
            hg[ch] = pltpu.async_copy(
                x_hbm.at[ridx_v.at[pl.ds(ch * _CH, _CH)]], bufs[slot], gsems[slot])
            if ch >= lag:
                _store(ch - lag)
        for c2 in range(_NCH - lag, _NCH):
            _store(c2)
        for c2 in range(_NCH - _NBUF, _NCH):
            hs[c2].wait()

    with jax.named_scope("mask"):
        for h in hm:
            h.wait()
        pltpu.sync_copy(mout_v, mout_hbm.at[pl.ds(out0, _RPW)])


@functools.cache
def _build_sc_prune(pair):
    return pl.kernel(
        functools.partial(_sc_prune_body, pair * 2 * _N),
        mesh=plsc.VectorSubcoreMesh(core_axis_name="c", subcore_axis_name="s"),
        out_type=(),
        scratch_types=[
            pltpu.VMEM((_TPW,), jnp.int32),
            pltpu.VMEM((_TPW // 128, 128), jnp.int32),
            pltpu.VMEM((_TPW // 128, 128), jnp.int32),
            pltpu.VMEM_SHARED((_N,), jnp.int32),
            pltpu.VMEM((_RPW,), jnp.int32),
            pltpu.VMEM((_RPW,), jnp.float32),
        ] + [pltpu.VMEM((_CH, _C), jnp.float32) for _ in range(_NBUF)]
          + [pltpu.SemaphoreType.DMA for _ in range(2 * _NBUF + 1)],
    )


def kernel(x, m, scores):
    x_flat = x.reshape(_B * _N, _C)
    m_flat = m.reshape(_B * _N)
    r01 = _rank_call(scores[:2].reshape(2, 1, _N)).reshape(2 * _N)
    r23 = _rank_call(scores[2:].reshape(2, 1, _N)).reshape(2 * _N)
    xout_ref = pl.empty_ref_like(pltpu.HBM((_B * _K, _C), jnp.float32))
    mout_ref = pl.empty_ref_like(pltpu.HBM((_B * _K,), jnp.float32))
    _build_sc_prune(0)(r01, x_flat, m_flat, xout_ref, mout_ref)
    _build_sc_prune(1)(r23, x_flat, m_flat, xout_ref, mout_ref)
    return (xout_ref[...].reshape(_B, _K, _C),
            mout_ref[...].reshape(_B, 1, 1, _K))
